# baseline (speedup 1.0000x reference)
import functools

import jax
import jax.numpy as jnp
from jax import lax
from jax.experimental import pallas as pl
from jax.experimental.pallas import tpu as pltpu
from jax.experimental.pallas import tpu_sc as plsc

EMBED_DIM = 64
BATCH = 16384
N_PROPS = 26

NUM_CORES = 2
NUM_SUBCORES = 16
NUM_WORKERS = NUM_CORES * NUM_SUBCORES
B_PER_W = BATCH // NUM_WORKERS
CHUNK = 128
N_CHUNKS = B_PER_W // CHUNK
LANES = 16


def _sc_body(xf_hbm, w_hbm, out_hbm, xs_v, idx_v, rows_v, sem):
    wid = lax.axis_index("s") * NUM_CORES + lax.axis_index("c")
    base = wid * B_PER_W

    pltpu.sync_copy(xf_hbm.at[pl.ds(base * N_PROPS, B_PER_W * N_PROPS)], xs_v)

    lane = lax.iota(jnp.int32, LANES) * N_PROPS
    groups_per_chunk = CHUNK // LANES
    for j in range(B_PER_W // LANES):
        vals = plsc.load_gather(xs_v, [lane + j * (LANES * N_PROPS)])
        idx_v[j // groups_per_chunk, pl.ds((j % groups_per_chunk) * LANES, LANES)] = vals

    copies = [
        pltpu.async_copy(
            w_hbm.at[idx_v.at[r]],
            rows_v.at[pl.ds(r * CHUNK, CHUNK)],
            sem,
        )
        for r in range(N_CHUNKS)
    ]
    for c in copies:
        c.wait()

    pltpu.sync_copy(rows_v, out_hbm.at[pl.ds(base, B_PER_W)])


@jax.jit
def kernel(x, W):
    mesh = plsc.VectorSubcoreMesh(core_axis_name="c", subcore_axis_name="s")
    run = functools.partial(
        pl.kernel,
        mesh=mesh,
        compiler_params=pltpu.CompilerParams(
            needs_layout_passes=False, use_tc_tiling_on_sc=False
        ),
        out_type=jax.ShapeDtypeStruct((BATCH, EMBED_DIM), jnp.float32),
        scratch_types=[
            pltpu.VMEM((B_PER_W * N_PROPS,), jnp.int32),
            pltpu.VMEM((N_CHUNKS, CHUNK), jnp.int32),
            pltpu.VMEM((B_PER_W, EMBED_DIM), jnp.float32),
            pltpu.SemaphoreType.DMA,
        ],
    )(_sc_body)
    return run(x.reshape(-1), W)
